# Initial kernel scaffold; baseline (speedup 1.0000x reference)
import jax
import jax.numpy as jnp
from jax.experimental import pallas as pl


def kernel(x, edge_index, W1l, b1l, W1r, W2l, b2l, W2r):
    raise NotImplementedError("write your pallas kernel here")


# speedup vs baseline: 6.6661x; 6.6661x over previous
import functools

import jax
import jax.numpy as jnp
from jax import lax
from jax.experimental import pallas as pl
from jax.experimental.pallas import tpu as pltpu
from jax.experimental.pallas import tpu_sc as plsc

N_NODES = 10000
N_EDGES = 320000
D = 128

NC = 2
NS = 16
NW = NC * NS

CH = 128
NBLK_E = N_EDGES // CH
ITER_E = -(-NBLK_E // NW)

RB = 80
NBLK_R = N_NODES // RB
ITER_R = -(-NBLK_R // NS)


def _make_sc_agg():
  mesh = plsc.VectorSubcoreMesh(core_axis_name="c", subcore_axis_name="s")

  @functools.partial(
      pl.kernel,
      out_type=(
          jax.ShapeDtypeStruct((NC, N_NODES, D), jnp.float32),
          jax.ShapeDtypeStruct((N_NODES,), jnp.float32),
          jax.ShapeDtypeStruct((N_NODES,), jnp.float32),
      ),
      mesh=mesh,
      scratch_types=[
          pltpu.VMEM((CH,), jnp.int32),
          pltpu.VMEM((CH,), jnp.int32),
          pltpu.VMEM((CH, D), jnp.float32),
          pltpu.VMEM((CH,), jnp.float32),
          pltpu.VMEM((RB, D), jnp.float32),
          pltpu.VMEM((N_NODES,), jnp.float32),
          pltpu.VMEM_SHARED((N_NODES, D), jnp.float32),
          pltpu.VMEM_SHARED((N_NODES,), jnp.float32),
          pltpu.SemaphoreType.DMA,
      ],
  )
  def sc_agg(x_hbm, src_hbm, dst_hbm, agg_out, cnt0_out, cnt1_out,
             srcv, dstv, rows, ones, zbuf, czero, agg_sh, cnt_sh, sem):
    c = lax.axis_index("c")
    s = lax.axis_index("s")
    wid = s * NC + c

    zero16 = jnp.zeros((16,), jnp.float32)
    one16 = jnp.ones((16,), jnp.float32)

    def fill_ones(i, carry):
      ones[pl.ds(i * 16, 16)] = one16
      return carry
    lax.fori_loop(0, CH // 16, fill_ones, 0)

    def fill_zb(i, carry):
      for cc in range(8):
        zbuf[i, pl.ds(cc * 16, 16)] = zero16
      return carry
    lax.fori_loop(0, RB, fill_zb, 0)

    def zero_agg(k, carry):
      blk = k * NS + s

      @pl.when(blk < NBLK_R)
      def _():
        pltpu.sync_copy(zbuf, agg_sh.at[pl.ds(blk * RB, RB)])
      return carry
    lax.fori_loop(0, ITER_R, zero_agg, 0)

    @pl.when(s == 0)
    def _():
      def fill_cz(i, carry):
        czero[pl.ds(i * 16, 16)] = zero16
        return carry
      lax.fori_loop(0, N_NODES // 16, fill_cz, 0)
      pltpu.sync_copy(czero, cnt_sh)

    plsc.subcore_barrier()

    def chunk(k, carry):
      blk = k * NW + wid

      @pl.when(blk < NBLK_E)
      def _():
        off = blk * CH
        pltpu.sync_copy(src_hbm.at[pl.ds(off, CH)], srcv)
        pltpu.sync_copy(dst_hbm.at[pl.ds(off, CH)], dstv)
        pltpu.async_copy(x_hbm.at[srcv], rows, sem).wait()
        pltpu.sync_copy(rows, agg_sh.at[dstv], add=True)
        pltpu.sync_copy(ones, cnt_sh.at[dstv], add=True)
      return carry
    lax.fori_loop(0, ITER_E, chunk, 0)

    plsc.subcore_barrier()

    def writeback(k, carry):
      blk = k * NS + s

      @pl.when(blk < NBLK_R)
      def _():
        pltpu.sync_copy(agg_sh.at[pl.ds(blk * RB, RB)],
                        agg_out.at[c, pl.ds(blk * RB, RB)])
      return carry
    lax.fori_loop(0, ITER_R, writeback, 0)

    @pl.when(s == 0)
    def _():
      @pl.when(c == 0)
      def _():
        pltpu.sync_copy(cnt_sh, cnt0_out)

      @pl.when(c == 1)
      def _():
        pltpu.sync_copy(cnt_sh, cnt1_out)

  return sc_agg


_sc_agg = _make_sc_agg()

BLK = 1000


def _tc_layer_body(agg_ref, cnt0_ref, cnt1_ref, x_ref, wl_ref, wr_ref, b_ref,
                   o_ref):
  agg = agg_ref[0] + agg_ref[1]
  cnt = cnt0_ref[...] + cnt1_ref[...]
  mean = agg / jnp.maximum(cnt, 1.0)
  h = lax.dot_general(mean, wl_ref[...], (((1,), (1,)), ((), ())),
                      preferred_element_type=jnp.float32)
  h = h + lax.dot_general(x_ref[...], wr_ref[...], (((1,), (1,)), ((), ())),
                          preferred_element_type=jnp.float32)
  o_ref[...] = jnp.maximum(h + b_ref[...], 0.0)


def _tc_layer(agg, cnt0, cnt1, x, Wl, bl, Wr):
  grid = (N_NODES // BLK,)
  return pl.pallas_call(
      _tc_layer_body,
      grid=grid,
      in_specs=[
          pl.BlockSpec((NC, BLK, D), lambda i: (0, i, 0)),
          pl.BlockSpec((BLK, 1), lambda i: (i, 0)),
          pl.BlockSpec((BLK, 1), lambda i: (i, 0)),
          pl.BlockSpec((BLK, D), lambda i: (i, 0)),
          pl.BlockSpec((D, D), lambda i: (0, 0)),
          pl.BlockSpec((D, D), lambda i: (0, 0)),
          pl.BlockSpec((1, D), lambda i: (0, 0)),
      ],
      out_specs=pl.BlockSpec((BLK, D), lambda i: (i, 0)),
      out_shape=jax.ShapeDtypeStruct((N_NODES, D), jnp.float32),
  )(agg, cnt0.reshape(N_NODES, 1), cnt1.reshape(N_NODES, 1), x, Wl, Wr,
    bl.reshape(1, D))


def kernel(x, edge_index, W1l, b1l, W1r, W2l, b2l, W2r):
  src = edge_index[0].astype(jnp.int32)
  dst = edge_index[1].astype(jnp.int32)
  x = x.astype(jnp.float32)

  agg1, cnt0, cnt1 = _sc_agg(x, src, dst)
  h = _tc_layer(agg1, cnt0, cnt1, x, W1l, b1l, W1r)
  agg2, _c0, _c1 = _sc_agg(h, src, dst)
  out = _tc_layer(agg2, cnt0, cnt1, h, W2l, b2l, W2r)
  return out


# speedup vs baseline: 13.6345x; 2.0453x over previous
import functools

import jax
import jax.numpy as jnp
from jax import lax
from jax.experimental import pallas as pl
from jax.experimental.pallas import tpu as pltpu
from jax.experimental.pallas import tpu_sc as plsc

N_NODES = 10000
N_EDGES = 320000
D = 128

NC = 2
NS = 16
NW = NC * NS

CH = 128
BPW = 80
NBLK_E = NW * BPW
E_PAD = NBLK_E * CH
N_PAD = E_PAD - N_EDGES
N_ACC = 10080

RB = 80
NBLK_Z = N_ACC // RB
NBLK_W = N_NODES // RB
ITER_R = -(-NBLK_Z // NS)


def _make_sc_agg():
  mesh = plsc.VectorSubcoreMesh(core_axis_name="c", subcore_axis_name="s")

  out_type = (
      jax.ShapeDtypeStruct((NC, N_NODES, D), jnp.float32),
      jax.ShapeDtypeStruct((N_ACC,), jnp.float32),
      jax.ShapeDtypeStruct((N_ACC,), jnp.float32),
  )
  scratch = [
      [pltpu.VMEM((CH,), jnp.int32) for _ in range(4)],
      [pltpu.VMEM((CH,), jnp.int32) for _ in range(4)],
      [pltpu.VMEM((CH, D), jnp.float32) for _ in range(2)],
      pltpu.VMEM((RB, D), jnp.float32),
      pltpu.VMEM((CH,), jnp.float32),
      pltpu.VMEM((RB,), jnp.float32),
      pltpu.VMEM_SHARED((N_ACC, D), jnp.float32),
      pltpu.VMEM_SHARED((N_ACC,), jnp.float32),
      [pltpu.SemaphoreType.DMA for _ in range(4)],
      [pltpu.SemaphoreType.DMA for _ in range(2)],
      [pltpu.SemaphoreType.DMA for _ in range(2)],
      [pltpu.SemaphoreType.DMA for _ in range(2)],
  ]

  @functools.partial(pl.kernel, out_type=out_type, mesh=mesh,
                     scratch_types=scratch)
  def sc_agg(x_hbm, src_hbm, dst_hbm, agg_out, cnt0_out, cnt1_out,
             srcv, dstv, rows, zbuf, ones, zcnt, agg_sh, cnt_sh,
             si, sg, ss, sc):
    c = lax.axis_index("c")
    s = lax.axis_index("s")
    wid = s * NC + c
    base = wid * BPW

    def issue_idx(j, t):
      off = (base + j) * CH
      pltpu.async_copy(src_hbm.at[pl.ds(off, CH)], srcv[t], si[t])
      pltpu.async_copy(dst_hbm.at[pl.ds(off, CH)], dstv[t], si[t])

    def wait_idx(j, t):
      off = (base + j) * CH
      pltpu.make_async_copy(src_hbm.at[pl.ds(off, CH)], srcv[t],
                            si[t]).wait()
      pltpu.make_async_copy(dst_hbm.at[pl.ds(off, CH)], dstv[t],
                            si[t]).wait()

    def issue_gather(j, t, b):
      pltpu.async_copy(x_hbm.at[srcv[t]], rows[b], sg[b])

    for u in range(4):
      issue_idx(u, u)

    zero16 = jnp.zeros((16,), jnp.float32)
    one16 = jnp.ones((16,), jnp.float32)

    def fill_zb(i, carry):
      for cc in range(8):
        zbuf[i, pl.ds(cc * 16, 16)] = zero16
      return carry
    lax.fori_loop(0, RB, fill_zb, 0)

    def fill_small(i, carry):
      ones[pl.ds(i * 16, 16)] = one16
      zcnt[pl.ds(i * 16, 16)] = zero16
      return carry
    lax.fori_loop(0, RB // 16, fill_small, 0)
    for i in range(RB // 16, CH // 16):
      ones[pl.ds(i * 16, 16)] = one16

    def zero_blocks(k, carry):
      blk = k * NS + s

      @pl.when(blk < NBLK_Z)
      def _():
        pltpu.sync_copy(zbuf, agg_sh.at[pl.ds(blk * RB, RB)])
        pltpu.sync_copy(zcnt, cnt_sh.at[pl.ds(blk * RB, RB)])
      return carry
    lax.fori_loop(0, ITER_R, zero_blocks, 0)

    wait_idx(0, 0)
    issue_gather(0, 0, 0)
    wait_idx(1, 1)
    issue_gather(1, 1, 1)

    plsc.subcore_barrier()

    def quad(p, carry):
      for u in range(4):
        j = p * 4 + u
        b = u % 2
        t = u

        pltpu.make_async_copy(x_hbm.at[srcv[t]], rows[b], sg[b]).wait()
        pltpu.async_copy(rows[b], agg_sh.at[dstv[t]], ss[b], add=True)
        pltpu.async_copy(ones, cnt_sh.at[dstv[t]], sc[b], add=True)

        pltpu.make_async_copy(rows[b], agg_sh.at[dstv[t]], ss[b]).wait()

        @pl.when(j + 2 < BPW)
        def _():
          t2 = (u + 2) % 4
          wait_idx(j + 2, t2)
          issue_gather(j + 2, t2, b)

        pltpu.make_async_copy(ones, cnt_sh.at[dstv[t]], sc[b]).wait()

        @pl.when(j + 4 < BPW)
        def _():
          issue_idx(j + 4, t)
      return carry
    lax.fori_loop(0, BPW // 4, quad, 0)

    plsc.subcore_barrier()

    def writeback(k, carry):
      blk = k * NS + s

      @pl.when(blk < NBLK_W)
      def _():
        pltpu.sync_copy(agg_sh.at[pl.ds(blk * RB, RB)],
                        agg_out.at[c, pl.ds(blk * RB, RB)])
      return carry
    lax.fori_loop(0, ITER_R, writeback, 0)

    @pl.when(s == 0)
    def _():
      @pl.when(c == 0)
      def _():
        pltpu.sync_copy(cnt_sh, cnt0_out)

      @pl.when(c == 1)
      def _():
        pltpu.sync_copy(cnt_sh, cnt1_out)

  return sc_agg


_sc_agg_cnt = _make_sc_agg()

BLK = 1000


def _tc_layer_body(agg_ref, cnt0_ref, cnt1_ref, x_ref, wl_ref, wr_ref, b_ref,
                   o_ref):
  agg = agg_ref[0] + agg_ref[1]
  cnt = cnt0_ref[...] + cnt1_ref[...]
  mean = agg / jnp.maximum(cnt, 1.0)
  h = lax.dot_general(mean, wl_ref[...], (((1,), (1,)), ((), ())),
                      preferred_element_type=jnp.float32)
  h = h + lax.dot_general(x_ref[...], wr_ref[...], (((1,), (1,)), ((), ())),
                          preferred_element_type=jnp.float32)
  o_ref[...] = jnp.maximum(h + b_ref[...], 0.0)


def _tc_layer(agg, cnt0, cnt1, x, Wl, bl, Wr):
  grid = (N_NODES // BLK,)
  return pl.pallas_call(
      _tc_layer_body,
      grid=grid,
      in_specs=[
          pl.BlockSpec((NC, BLK, D), lambda i: (0, i, 0)),
          pl.BlockSpec((BLK, 1), lambda i: (i, 0)),
          pl.BlockSpec((BLK, 1), lambda i: (i, 0)),
          pl.BlockSpec((BLK, D), lambda i: (i, 0)),
          pl.BlockSpec((D, D), lambda i: (0, 0)),
          pl.BlockSpec((D, D), lambda i: (0, 0)),
          pl.BlockSpec((1, D), lambda i: (0, 0)),
      ],
      out_specs=pl.BlockSpec((BLK, D), lambda i: (i, 0)),
      out_shape=jax.ShapeDtypeStruct((N_NODES, D), jnp.float32),
  )(agg, cnt0.reshape(N_NODES, 1), cnt1.reshape(N_NODES, 1), x, Wl, Wr,
    bl.reshape(1, D))


def kernel(x, edge_index, W1l, b1l, W1r, W2l, b2l, W2r):
  src = edge_index[0].astype(jnp.int32)
  dst = edge_index[1].astype(jnp.int32)
  x = x.astype(jnp.float32)

  pad_src = jnp.arange(N_PAD, dtype=jnp.int32) % N_NODES
  pad_dst = N_NODES + jnp.arange(N_PAD, dtype=jnp.int32) % RB
  src_p = jnp.concatenate([src, pad_src])
  dst_p = jnp.concatenate([dst, pad_dst])

  agg1, cnt0, cnt1 = _sc_agg_cnt(x, src_p, dst_p)
  cnt0 = cnt0[:N_NODES]
  cnt1 = cnt1[:N_NODES]
  h = _tc_layer(agg1, cnt0, cnt1, x, W1l, b1l, W1r)
  agg2, _c0, _c1 = _sc_agg_cnt(h, src_p, dst_p)
  out = _tc_layer(agg2, cnt0, cnt1, h, W2l, b2l, W2r)
  return out


# speedup vs baseline: 13.6698x; 1.0026x over previous
import functools

import jax
import jax.numpy as jnp
from jax import lax
from jax.experimental import pallas as pl
from jax.experimental.pallas import tpu as pltpu
from jax.experimental.pallas import tpu_sc as plsc

N_NODES = 10000
N_EDGES = 320000
D = 128

NC = 2
NS = 16
NW = NC * NS

CH = 128
BPW = 80
NREAL = N_EDGES // CH
N_ACC = N_NODES

RB = 80
NBLK_Z = N_ACC // RB
NBLK_W = N_NODES // RB
ITER_R = -(-NBLK_Z // NS)


def _make_sc_agg():
  mesh = plsc.VectorSubcoreMesh(core_axis_name="c", subcore_axis_name="s")

  out_type = (
      jax.ShapeDtypeStruct((NC, N_NODES, D), jnp.float32),
      jax.ShapeDtypeStruct((N_ACC,), jnp.float32),
      jax.ShapeDtypeStruct((N_ACC,), jnp.float32),
  )
  scratch = [
      [pltpu.VMEM((CH,), jnp.int32) for _ in range(4)],
      [pltpu.VMEM((CH,), jnp.int32) for _ in range(4)],
      [pltpu.VMEM((CH, D), jnp.float32) for _ in range(3)],
      pltpu.VMEM((CH,), jnp.float32),
      pltpu.VMEM((RB,), jnp.float32),
      pltpu.VMEM_SHARED((N_ACC, D), jnp.float32),
      pltpu.VMEM_SHARED((N_ACC,), jnp.float32),
      [pltpu.SemaphoreType.DMA for _ in range(4)],
      [pltpu.SemaphoreType.DMA for _ in range(3)],
      [pltpu.SemaphoreType.DMA for _ in range(3)],
      [pltpu.SemaphoreType.DMA for _ in range(3)],
  ]

  @functools.partial(pl.kernel, out_type=out_type, mesh=mesh,
                     scratch_types=scratch)
  def sc_agg(x_hbm, src_hbm, dst_hbm, zeros_hbm, agg_out, cnt0_out, cnt1_out,
             srcv, dstv, rows, ones, zcnt, agg_sh, cnt_sh, si, sg, ss, sc):
    c = lax.axis_index("c")
    s = lax.axis_index("s")
    wid = s * NC + c
    base = wid * BPW

    def issue_idx(j, t):
      off = (base + j) * CH
      pltpu.async_copy(src_hbm.at[pl.ds(off, CH)], srcv[t], si[t])
      pltpu.async_copy(dst_hbm.at[pl.ds(off, CH)], dstv[t], si[t])

    def wait_idx(t):
      pltpu.make_async_copy(src_hbm.at[pl.ds(0, CH)], srcv[t], si[t]).wait()
      pltpu.make_async_copy(dst_hbm.at[pl.ds(0, CH)], dstv[t], si[t]).wait()

    def issue_gather(t, b):
      pltpu.async_copy(x_hbm.at[srcv[t]], rows[b], sg[b])

    def wait_gather(t, b):
      pltpu.make_async_copy(x_hbm.at[srcv[t]], rows[b], sg[b]).wait()

    def issue_scatter(t, b):
      pltpu.async_copy(rows[b], agg_sh.at[dstv[t]], ss[b], add=True)
      pltpu.async_copy(ones, cnt_sh.at[dstv[t]], sc[b], add=True)

    def wait_scatter(t, b):
      pltpu.make_async_copy(rows[b], agg_sh.at[dstv[t]], ss[b]).wait()
      pltpu.make_async_copy(ones, cnt_sh.at[dstv[t]], sc[b]).wait()

    for u in range(3):
      issue_idx(u, u)

    zero16 = jnp.zeros((16,), jnp.float32)
    one16 = jnp.ones((16,), jnp.float32)
    for i in range(CH // 16):
      ones[pl.ds(i * 16, 16)] = one16
    for i in range(RB // 16):
      zcnt[pl.ds(i * 16, 16)] = zero16

    def zero_blocks(k, carry):
      blk = k * NS + s

      @pl.when(blk < NBLK_Z)
      def _():
        pltpu.sync_copy(zeros_hbm.at[pl.ds(blk * RB, RB)],
                        agg_sh.at[pl.ds(blk * RB, RB)])
        pltpu.sync_copy(zcnt, cnt_sh.at[pl.ds(blk * RB, RB)])
      return carry
    lax.fori_loop(0, ITER_R, zero_blocks, 0)

    wait_idx(0)
    issue_gather(0, 0)
    wait_idx(1)
    issue_gather(1, 1)

    plsc.subcore_barrier()

    def slot(j, u, first, g2, g3):
      b = u % 3
      b2 = (u + 2) % 3
      t = u % 4
      t2 = (u + 2) % 4
      tn = (u + 3) % 4

      @pl.when(base + j < NREAL)
      def _():
        wait_gather(t, b)
        issue_scatter(t, b)

      if not first:
        @pl.when(base + j - 1 < NREAL)
        def _():
          wait_scatter(tn, b2)

      if g2:
        @pl.when(base + j + 2 < NREAL)
        def _():
          wait_idx(t2)
          issue_gather(t2, b2)

      if g3:
        @pl.when(base + j + 3 < NREAL)
        def _():
          issue_idx(j + 3, tn)

    for u in range(12):
      slot(u, u, u == 0, True, True)

    def body(p, carry):
      j0 = p * 12
      for u in range(12):
        slot(j0 + u, u, False, True, True)
      return carry
    lax.fori_loop(1, 6, body, 0)

    for u in range(8):
      j = 72 + u
      slot(j, u, False, j + 2 < BPW, j + 3 < BPW)

    @pl.when(base + 79 < NREAL)
    def _():
      wait_scatter(79 % 4, 79 % 3)

    plsc.subcore_barrier()

    def writeback(k, carry):
      blk = k * NS + s

      @pl.when(blk < NBLK_W)
      def _():
        pltpu.sync_copy(agg_sh.at[pl.ds(blk * RB, RB)],
                        agg_out.at[c, pl.ds(blk * RB, RB)])
      return carry
    lax.fori_loop(0, ITER_R, writeback, 0)

    @pl.when(s == 0)
    def _():
      @pl.when(c == 0)
      def _():
        pltpu.sync_copy(cnt_sh, cnt0_out)

      @pl.when(c == 1)
      def _():
        pltpu.sync_copy(cnt_sh, cnt1_out)

  return sc_agg


_sc_agg_cnt = _make_sc_agg()

BLK = 1000


def _tc_right_body(x_ref, wr_ref, o_ref):
  o_ref[...] = lax.dot_general(x_ref[...], wr_ref[...],
                               (((1,), (1,)), ((), ())),
                               preferred_element_type=jnp.float32)


def _tc_right(x, Wr):
  return pl.pallas_call(
      _tc_right_body,
      grid=(N_NODES // BLK,),
      in_specs=[
          pl.BlockSpec((BLK, D), lambda i: (i, 0)),
          pl.BlockSpec((D, D), lambda i: (0, 0)),
      ],
      out_specs=pl.BlockSpec((BLK, D), lambda i: (i, 0)),
      out_shape=jax.ShapeDtypeStruct((N_NODES, D), jnp.float32),
  )(x, Wr)


def _tc_combine_body(agg_ref, cnt0_ref, cnt1_ref, xr_ref, wl_ref, b_ref,
                     o_ref):
  agg = agg_ref[0] + agg_ref[1]
  cnt = cnt0_ref[...] + cnt1_ref[...]
  mean = agg / jnp.maximum(cnt, 1.0)
  h = lax.dot_general(mean, wl_ref[...], (((1,), (1,)), ((), ())),
                      preferred_element_type=jnp.float32)
  o_ref[...] = jnp.maximum(h + xr_ref[...] + b_ref[...], 0.0)


def _tc_combine(agg, cnt0, cnt1, xr, Wl, bl):
  return pl.pallas_call(
      _tc_combine_body,
      grid=(N_NODES // BLK,),
      in_specs=[
          pl.BlockSpec((NC, BLK, D), lambda i: (0, i, 0)),
          pl.BlockSpec((BLK, 1), lambda i: (i, 0)),
          pl.BlockSpec((BLK, 1), lambda i: (i, 0)),
          pl.BlockSpec((BLK, D), lambda i: (i, 0)),
          pl.BlockSpec((D, D), lambda i: (0, 0)),
          pl.BlockSpec((1, D), lambda i: (0, 0)),
      ],
      out_specs=pl.BlockSpec((BLK, D), lambda i: (i, 0)),
      out_shape=jax.ShapeDtypeStruct((N_NODES, D), jnp.float32),
  )(agg, cnt0.reshape(N_NODES, 1), cnt1.reshape(N_NODES, 1), xr, Wl,
    bl.reshape(1, D))


def kernel(x, edge_index, W1l, b1l, W1r, W2l, b2l, W2r):
  src = edge_index[0].astype(jnp.int32)
  dst = edge_index[1].astype(jnp.int32)
  x = x.astype(jnp.float32)

  zeros = jnp.zeros((N_ACC, D), jnp.float32)

  agg1, cnt0, cnt1 = _sc_agg_cnt(x, src, dst, zeros)
  xr1 = _tc_right(x, W1r)
  h = _tc_combine(agg1, cnt0, cnt1, xr1, W1l, b1l)
  agg2, _c0, _c1 = _sc_agg_cnt(h, src, dst, zeros)
  xr2 = _tc_right(h, W2r)
  out = _tc_combine(agg2, cnt0, cnt1, xr2, W2l, b2l)
  return out


# speedup vs baseline: 14.5398x; 1.0636x over previous
import functools

import jax
import jax.numpy as jnp
from jax import lax
from jax.experimental import pallas as pl
from jax.experimental.pallas import tpu as pltpu
from jax.experimental.pallas import tpu_sc as plsc

N_NODES = 10000
N_EDGES = 320000
D = 128

NC = 2
NS = 16
NW = NC * NS

CH = 128
BPW = 80
NREAL = N_EDGES // CH
N_ACC = N_NODES

RB = 80
NBLK_Z = N_ACC // RB
NBLK_W = N_NODES // RB
ITER_R = -(-NBLK_Z // NS)


def _make_sc_agg():
  mesh = plsc.VectorSubcoreMesh(core_axis_name="c", subcore_axis_name="s")

  out_type = (
      jax.ShapeDtypeStruct((NC, N_NODES, D), jnp.float32),
      jax.ShapeDtypeStruct((N_ACC,), jnp.float32),
      jax.ShapeDtypeStruct((N_ACC,), jnp.float32),
  )
  scratch = [
      [pltpu.VMEM((2, CH), jnp.int32) for _ in range(4)],
      [pltpu.VMEM((CH, D), jnp.float32) for _ in range(3)],
      pltpu.VMEM((CH,), jnp.float32),
      pltpu.VMEM((RB,), jnp.float32),
      pltpu.VMEM_SHARED((N_ACC, D), jnp.float32),
      pltpu.VMEM_SHARED((N_ACC,), jnp.float32),
      [pltpu.SemaphoreType.DMA for _ in range(4)],
      [pltpu.SemaphoreType.DMA for _ in range(3)],
      [pltpu.SemaphoreType.DMA for _ in range(3)],
      [pltpu.SemaphoreType.DMA for _ in range(3)],
      pltpu.SemaphoreType.DMA,
      pltpu.SemaphoreType.DMA,
  ]

  @functools.partial(pl.kernel, out_type=out_type, mesh=mesh,
                     scratch_types=scratch)
  def sc_agg(x_hbm, epack_hbm, zeros_hbm, agg_out, cnt0_out, cnt1_out,
             idxv, rows, ones, zcnt, agg_sh, cnt_sh, si, sg, ss, sc,
             sz, sw):
    c = lax.axis_index("c")
    s = lax.axis_index("s")
    wid = s * NC + c
    base = wid * BPW

    def issue_idx(j, t):
      pltpu.async_copy(epack_hbm.at[base + j], idxv[t], si[t])

    def wait_idx(t):
      pltpu.make_async_copy(epack_hbm.at[0], idxv[t], si[t]).wait()

    def issue_gather(t, b):
      pltpu.async_copy(x_hbm.at[idxv[t].at[0]], rows[b], sg[b])

    def wait_gather(t, b):
      pltpu.make_async_copy(x_hbm.at[idxv[t].at[0]], rows[b], sg[b]).wait()

    def issue_scatter(t, b):
      pltpu.async_copy(rows[b], agg_sh.at[idxv[t].at[1]], ss[b], add=True)
      pltpu.async_copy(ones, cnt_sh.at[idxv[t].at[1]], sc[b], add=True)

    def wait_scatter(t, b):
      pltpu.make_async_copy(rows[b], agg_sh.at[idxv[t].at[1]], ss[b]).wait()
      pltpu.make_async_copy(ones, cnt_sh.at[idxv[t].at[1]], sc[b]).wait()

    for u in range(3):
      issue_idx(u, u)

    zero16 = jnp.zeros((16,), jnp.float32)
    one16 = jnp.ones((16,), jnp.float32)
    for i in range(CH // 16):
      ones[pl.ds(i * 16, 16)] = one16
    for i in range(RB // 16):
      zcnt[pl.ds(i * 16, 16)] = zero16

    def zero_blocks(k, carry):
      blk = k * NS + s

      @pl.when(blk < NBLK_Z)
      def _():
        pltpu.async_copy(zeros_hbm.at[pl.ds(blk * RB, RB)],
                         agg_sh.at[pl.ds(blk * RB, RB)], sz)
        pltpu.async_copy(zcnt, cnt_sh.at[pl.ds(blk * RB, RB)], sz)
      return carry
    lax.fori_loop(0, ITER_R, zero_blocks, 0)

    wait_idx(0)
    issue_gather(0, 0)
    wait_idx(1)
    issue_gather(1, 1)

    def zero_wait(k, carry):
      blk = k * NS + s

      @pl.when(blk < NBLK_Z)
      def _():
        pltpu.make_async_copy(zeros_hbm.at[pl.ds(blk * RB, RB)],
                              agg_sh.at[pl.ds(blk * RB, RB)], sz).wait()
        pltpu.make_async_copy(zcnt, cnt_sh.at[pl.ds(blk * RB, RB)],
                              sz).wait()
      return carry
    lax.fori_loop(0, ITER_R, zero_wait, 0)

    plsc.subcore_barrier()

    def slot(j, u, first, g2, g3):
      b = u % 3
      b2 = (u + 2) % 3
      t = u % 4
      t2 = (u + 2) % 4
      tn = (u + 3) % 4

      @pl.when(base + j < NREAL)
      def _():
        wait_gather(t, b)
        issue_scatter(t, b)

      if not first:
        @pl.when(base + j - 1 < NREAL)
        def _():
          wait_scatter(tn, b2)

      if g2:
        @pl.when(base + j + 2 < NREAL)
        def _():
          wait_idx(t2)
          issue_gather(t2, b2)

      if g3:
        @pl.when(base + j + 3 < NREAL)
        def _():
          issue_idx(j + 3, tn)

    for u in range(12):
      slot(u, u, u == 0, True, True)

    def body(p, carry):
      j0 = p * 12
      for u in range(12):
        slot(j0 + u, u, False, True, True)
      return carry
    lax.fori_loop(1, 6, body, 0)

    for u in range(8):
      j = 72 + u
      slot(j, u, False, j + 2 < BPW, j + 3 < BPW)

    @pl.when(base + 79 < NREAL)
    def _():
      wait_scatter(79 % 4, 79 % 3)

    plsc.subcore_barrier()

    def writeback(k, carry):
      blk = k * NS + s

      @pl.when(blk < NBLK_W)
      def _():
        pltpu.async_copy(agg_sh.at[pl.ds(blk * RB, RB)],
                         agg_out.at[c, pl.ds(blk * RB, RB)], sw)
      return carry
    lax.fori_loop(0, ITER_R, writeback, 0)

    def writeback_wait(k, carry):
      blk = k * NS + s

      @pl.when(blk < NBLK_W)
      def _():
        pltpu.make_async_copy(agg_sh.at[pl.ds(blk * RB, RB)],
                              agg_out.at[c, pl.ds(blk * RB, RB)], sw).wait()
      return carry
    lax.fori_loop(0, ITER_R, writeback_wait, 0)

    @pl.when(s == 0)
    def _():
      @pl.when(c == 0)
      def _():
        pltpu.sync_copy(cnt_sh, cnt0_out)

      @pl.when(c == 1)
      def _():
        pltpu.sync_copy(cnt_sh, cnt1_out)

  return sc_agg


_sc_agg_cnt = _make_sc_agg()

BLK = 1000


def _tc_right_body(x_ref, wr_ref, o_ref):
  o_ref[...] = lax.dot_general(x_ref[...], wr_ref[...],
                               (((1,), (1,)), ((), ())),
                               preferred_element_type=jnp.float32)


def _tc_right(x, Wr):
  return pl.pallas_call(
      _tc_right_body,
      grid=(N_NODES // BLK,),
      in_specs=[
          pl.BlockSpec((BLK, D), lambda i: (i, 0)),
          pl.BlockSpec((D, D), lambda i: (0, 0)),
      ],
      out_specs=pl.BlockSpec((BLK, D), lambda i: (i, 0)),
      out_shape=jax.ShapeDtypeStruct((N_NODES, D), jnp.float32),
  )(x, Wr)


def _tc_combine_body(agg_ref, cnt0_ref, cnt1_ref, xr_ref, wl_ref, b_ref,
                     o_ref):
  agg = agg_ref[0] + agg_ref[1]
  cnt = cnt0_ref[...] + cnt1_ref[...]
  mean = agg / jnp.maximum(cnt, 1.0)
  h = lax.dot_general(mean, wl_ref[...], (((1,), (1,)), ((), ())),
                      preferred_element_type=jnp.float32)
  o_ref[...] = jnp.maximum(h + xr_ref[...] + b_ref[...], 0.0)


def _tc_combine(agg, cnt0, cnt1, xr, Wl, bl):
  return pl.pallas_call(
      _tc_combine_body,
      grid=(N_NODES // BLK,),
      in_specs=[
          pl.BlockSpec((NC, BLK, D), lambda i: (0, i, 0)),
          pl.BlockSpec((BLK, 1), lambda i: (i, 0)),
          pl.BlockSpec((BLK, 1), lambda i: (i, 0)),
          pl.BlockSpec((BLK, D), lambda i: (i, 0)),
          pl.BlockSpec((D, D), lambda i: (0, 0)),
          pl.BlockSpec((1, D), lambda i: (0, 0)),
      ],
      out_specs=pl.BlockSpec((BLK, D), lambda i: (i, 0)),
      out_shape=jax.ShapeDtypeStruct((N_NODES, D), jnp.float32),
  )(agg, cnt0.reshape(N_NODES, 1), cnt1.reshape(N_NODES, 1), xr, Wl,
    bl.reshape(1, D))


def kernel(x, edge_index, W1l, b1l, W1r, W2l, b2l, W2r):
  src = edge_index[0].astype(jnp.int32)
  dst = edge_index[1].astype(jnp.int32)
  x = x.astype(jnp.float32)

  zeros = jnp.zeros((N_ACC, D), jnp.float32)
  epack = jnp.stack([src.reshape(NREAL, CH), dst.reshape(NREAL, CH)], axis=1)

  agg1, cnt0, cnt1 = _sc_agg_cnt(x, epack, zeros)
  xr1 = _tc_right(x, W1r)
  h = _tc_combine(agg1, cnt0, cnt1, xr1, W1l, b1l)
  agg2, _c0, _c1 = _sc_agg_cnt(h, epack, zeros)
  xr2 = _tc_right(h, W2r)
  out = _tc_combine(agg2, cnt0, cnt1, xr2, W2l, b2l)
  return out


# speedup vs baseline: 14.6774x; 1.0095x over previous
import functools

import jax
import jax.numpy as jnp
from jax import lax
from jax.experimental import pallas as pl
from jax.experimental.pallas import tpu as pltpu
from jax.experimental.pallas import tpu_sc as plsc

N_NODES = 10000
N_EDGES = 320000
D = 128

NC = 2
NS = 16
NW = NC * NS

CH = 128
BPW = 80
NREAL = N_EDGES // CH
N_ACC = N_NODES

RB = 80
NBLK_Z = N_ACC // RB
NBLK_W = N_NODES // RB
ITER_R = -(-NBLK_Z // NS)


def _make_sc_agg(with_cnt):
  mesh = plsc.VectorSubcoreMesh(core_axis_name="c", subcore_axis_name="s")

  out_type = [jax.ShapeDtypeStruct((NC, N_NODES, D), jnp.float32)]
  scratch = [
      [pltpu.VMEM((2, CH), jnp.int32) for _ in range(4)],
      [pltpu.VMEM((CH, D), jnp.float32) for _ in range(3)],
      pltpu.VMEM_SHARED((N_ACC, D), jnp.float32),
      [pltpu.SemaphoreType.DMA for _ in range(4)],
      [pltpu.SemaphoreType.DMA for _ in range(3)],
      [pltpu.SemaphoreType.DMA for _ in range(3)],
      pltpu.SemaphoreType.DMA,
      pltpu.SemaphoreType.DMA,
  ]
  if with_cnt:
    out_type += [jax.ShapeDtypeStruct((N_ACC,), jnp.float32),
                 jax.ShapeDtypeStruct((N_ACC,), jnp.float32)]
    scratch += [
        pltpu.VMEM((CH,), jnp.float32),
        pltpu.VMEM((RB,), jnp.float32),
        pltpu.VMEM_SHARED((N_ACC,), jnp.float32),
        [pltpu.SemaphoreType.DMA for _ in range(3)],
    ]

  @functools.partial(pl.kernel, out_type=tuple(out_type), mesh=mesh,
                     scratch_types=scratch)
  def sc_agg(x_hbm, epack_hbm, zeros_hbm, *refs):
    if with_cnt:
      (agg_out, cnt0_out, cnt1_out, idxv, rows, agg_sh, si, sg, ss,
       sz, sw, ones, zcnt, cnt_sh, sc) = refs
    else:
      (agg_out, idxv, rows, agg_sh, si, sg, ss, sz, sw) = refs
    c = lax.axis_index("c")
    s = lax.axis_index("s")
    wid = s * NC + c
    base = wid * BPW

    def issue_idx(j, t):
      pltpu.async_copy(epack_hbm.at[base + j], idxv[t], si[t])

    def wait_idx(t):
      pltpu.make_async_copy(epack_hbm.at[0], idxv[t], si[t]).wait()

    def issue_gather(t, b):
      pltpu.async_copy(x_hbm.at[idxv[t].at[0]], rows[b], sg[b])

    def wait_gather(t, b):
      pltpu.make_async_copy(x_hbm.at[idxv[t].at[0]], rows[b], sg[b]).wait()

    def issue_scatter(t, b):
      pltpu.async_copy(rows[b], agg_sh.at[idxv[t].at[1]], ss[b], add=True)
      if with_cnt:
        pltpu.async_copy(ones, cnt_sh.at[idxv[t].at[1]], sc[b], add=True)

    def wait_scatter(t, b):
      pltpu.make_async_copy(rows[b], agg_sh.at[idxv[t].at[1]], ss[b]).wait()
      if with_cnt:
        pltpu.make_async_copy(ones, cnt_sh.at[idxv[t].at[1]],
                              sc[b]).wait()

    for u in range(3):
      issue_idx(u, u)

    if with_cnt:
      zero16 = jnp.zeros((16,), jnp.float32)
      one16 = jnp.ones((16,), jnp.float32)
      for i in range(CH // 16):
        ones[pl.ds(i * 16, 16)] = one16
      for i in range(RB // 16):
        zcnt[pl.ds(i * 16, 16)] = zero16

    def zero_blocks(k, carry):
      blk = k * NS + s

      @pl.when(blk < NBLK_Z)
      def _():
        pltpu.async_copy(zeros_hbm.at[pl.ds(blk * RB, RB)],
                         agg_sh.at[pl.ds(blk * RB, RB)], sz)
        if with_cnt:
          pltpu.async_copy(zcnt, cnt_sh.at[pl.ds(blk * RB, RB)], sz)
      return carry
    lax.fori_loop(0, ITER_R, zero_blocks, 0)

    wait_idx(0)
    issue_gather(0, 0)
    wait_idx(1)
    issue_gather(1, 1)

    def zero_wait(k, carry):
      blk = k * NS + s

      @pl.when(blk < NBLK_Z)
      def _():
        pltpu.make_async_copy(zeros_hbm.at[pl.ds(blk * RB, RB)],
                              agg_sh.at[pl.ds(blk * RB, RB)], sz).wait()
        if with_cnt:
          pltpu.make_async_copy(zcnt, cnt_sh.at[pl.ds(blk * RB, RB)],
                                sz).wait()
      return carry
    lax.fori_loop(0, ITER_R, zero_wait, 0)

    plsc.subcore_barrier()

    def slot(j, u, first, g2, g3):
      b = u % 3
      b2 = (u + 2) % 3
      t = u % 4
      t2 = (u + 2) % 4
      tn = (u + 3) % 4

      @pl.when(base + j < NREAL)
      def _():
        wait_gather(t, b)
        issue_scatter(t, b)

      if not first:
        @pl.when(base + j - 1 < NREAL)
        def _():
          wait_scatter(tn, b2)

      if g2:
        @pl.when(base + j + 2 < NREAL)
        def _():
          wait_idx(t2)
          issue_gather(t2, b2)

      if g3:
        @pl.when(base + j + 3 < NREAL)
        def _():
          issue_idx(j + 3, tn)

    for u in range(12):
      slot(u, u, u == 0, True, True)

    def body(p, carry):
      j0 = p * 12
      for u in range(12):
        slot(j0 + u, u, False, True, True)
      return carry
    lax.fori_loop(1, 6, body, 0)

    for u in range(8):
      j = 72 + u
      slot(j, u, False, j + 2 < BPW, j + 3 < BPW)

    @pl.when(base + 79 < NREAL)
    def _():
      wait_scatter(79 % 4, 79 % 3)

    plsc.subcore_barrier()

    def writeback(k, carry):
      blk = k * NS + s

      @pl.when(blk < NBLK_W)
      def _():
        pltpu.async_copy(agg_sh.at[pl.ds(blk * RB, RB)],
                         agg_out.at[c, pl.ds(blk * RB, RB)], sw)
      return carry
    lax.fori_loop(0, ITER_R, writeback, 0)

    def writeback_wait(k, carry):
      blk = k * NS + s

      @pl.when(blk < NBLK_W)
      def _():
        pltpu.make_async_copy(agg_sh.at[pl.ds(blk * RB, RB)],
                              agg_out.at[c, pl.ds(blk * RB, RB)], sw).wait()
      return carry
    lax.fori_loop(0, ITER_R, writeback_wait, 0)

    if with_cnt:
      @pl.when(s == 0)
      def _():
        @pl.when(c == 0)
        def _():
          pltpu.sync_copy(cnt_sh, cnt0_out)

        @pl.when(c == 1)
        def _():
          pltpu.sync_copy(cnt_sh, cnt1_out)

  return sc_agg


_sc_agg_cnt = _make_sc_agg(True)
_sc_agg_nocnt = _make_sc_agg(False)

BLK = 1000


def _tc_right_body(x_ref, wr_ref, o_ref):
  o_ref[...] = lax.dot_general(x_ref[...], wr_ref[...],
                               (((1,), (1,)), ((), ())),
                               preferred_element_type=jnp.float32)


def _tc_right(x, Wr):
  return pl.pallas_call(
      _tc_right_body,
      grid=(N_NODES // BLK,),
      in_specs=[
          pl.BlockSpec((BLK, D), lambda i: (i, 0)),
          pl.BlockSpec((D, D), lambda i: (0, 0)),
      ],
      out_specs=pl.BlockSpec((BLK, D), lambda i: (i, 0)),
      out_shape=jax.ShapeDtypeStruct((N_NODES, D), jnp.float32),
  )(x, Wr)


def _tc_combine_body(agg_ref, cnt0_ref, cnt1_ref, xr_ref, wl_ref, b_ref,
                     o_ref):
  agg = agg_ref[0] + agg_ref[1]
  cnt = cnt0_ref[...] + cnt1_ref[...]
  mean = agg / jnp.maximum(cnt, 1.0)
  h = lax.dot_general(mean, wl_ref[...], (((1,), (1,)), ((), ())),
                      preferred_element_type=jnp.float32)
  o_ref[...] = jnp.maximum(h + xr_ref[...] + b_ref[...], 0.0)


def _tc_combine(agg, cnt0, cnt1, xr, Wl, bl):
  return pl.pallas_call(
      _tc_combine_body,
      grid=(N_NODES // BLK,),
      in_specs=[
          pl.BlockSpec((NC, BLK, D), lambda i: (0, i, 0)),
          pl.BlockSpec((BLK, 1), lambda i: (i, 0)),
          pl.BlockSpec((BLK, 1), lambda i: (i, 0)),
          pl.BlockSpec((BLK, D), lambda i: (i, 0)),
          pl.BlockSpec((D, D), lambda i: (0, 0)),
          pl.BlockSpec((1, D), lambda i: (0, 0)),
      ],
      out_specs=pl.BlockSpec((BLK, D), lambda i: (i, 0)),
      out_shape=jax.ShapeDtypeStruct((N_NODES, D), jnp.float32),
  )(agg, cnt0.reshape(N_NODES, 1), cnt1.reshape(N_NODES, 1), xr, Wl,
    bl.reshape(1, D))


def kernel(x, edge_index, W1l, b1l, W1r, W2l, b2l, W2r):
  src = edge_index[0].astype(jnp.int32)
  dst = edge_index[1].astype(jnp.int32)
  x = x.astype(jnp.float32)

  zeros = jnp.zeros((N_ACC, D), jnp.float32)
  epack = jnp.stack([src.reshape(NREAL, CH), dst.reshape(NREAL, CH)], axis=1)

  agg1, cnt0, cnt1 = _sc_agg_cnt(x, epack, zeros)
  xr1 = _tc_right(x, W1r)
  h = _tc_combine(agg1, cnt0, cnt1, xr1, W1l, b1l)
  (agg2,) = _sc_agg_nocnt(h, epack, zeros)
  xr2 = _tc_right(h, W2r)
  out = _tc_combine(agg2, cnt0, cnt1, xr2, W2l, b2l)
  return out
